# baseline (speedup 1.0000x reference)
import functools

import jax
import jax.numpy as jnp
from jax import lax
from jax.experimental import pallas as pl
from jax.experimental.pallas import tpu as pltpu
from jax.experimental.pallas import tpu_sc as plsc

N = 10000
E = 320000
NPAD = 10112
NC = 2
NS = 16
KB = 128
K = 64
NROWS = 160
HROWS = NROWS // 2
DEPTH = 4
DEG_SPLIT = 80
STRIPE = NPAD // NS

_mesh = plsc.VectorSubcoreMesh(core_axis_name="c", subcore_axis_name="s")


def _pack_edges(arr, pad_value):
    cap = NS * NROWS * KB
    arr = jnp.concatenate(
        [arr, jnp.full((cap - E,), pad_value, jnp.int32)])
    return arr.reshape(NS, NROWS, KB)


def _sc_degree(dstp, zeros16):
    @functools.partial(
        pl.kernel,
        out_type=jax.ShapeDtypeStruct((NC, NPAD, 16), jnp.float32),
        mesh=_mesh,
        scratch_types=[
            pltpu.VMEM((NROWS, KB), jnp.int32),
            pltpu.VMEM((KB, 16), jnp.float32),
            pltpu.VMEM_SHARED((NPAD, 16), jnp.float32),
            pltpu.SemaphoreType.DMA,
        ],
    )
    def deg_kernel(dst_hbm, zero_hbm, out_hbm, dst_v, ones_v, acc_sh, sem):
        cid = lax.axis_index("c")
        sid = lax.axis_index("s")

        pltpu.async_copy(dst_hbm.at[sid], dst_v, sem).wait()
        @pl.loop(0, KB)
        def _(r):
            ones_v[r, :] = jnp.full((16,), 1.0, jnp.float32)

        pltpu.sync_copy(zero_hbm, acc_sh.at[pl.ds(sid * STRIPE, STRIPE)])
        plsc.subcore_barrier()

        bounds = ((0, DEG_SPLIT), (DEG_SPLIT, NROWS))
        for c_i in range(NC):
            @pl.when(cid == c_i)
            def _(c_i=c_i):
                @pl.loop(bounds[c_i][0], bounds[c_i][1])
                def _(c):
                    pltpu.sync_copy(ones_v, acc_sh.at[dst_v.at[c]], add=True)

        plsc.subcore_barrier()
        pltpu.sync_copy(acc_sh.at[pl.ds(sid * STRIPE, STRIPE)],
                        out_hbm.at[cid, pl.ds(sid * STRIPE, STRIPE)])

    return deg_kernel(dstp, zeros16)


def _sc_propagate(hp, srcp, dstp, zeros):
    @functools.partial(
        pl.kernel,
        out_type=jax.ShapeDtypeStruct((NPAD, 128), jnp.float32),
        mesh=_mesh,
        scratch_types=[
            pltpu.VMEM((HROWS, KB), jnp.int32),
            pltpu.VMEM((DEPTH, K), jnp.int32),
            pltpu.VMEM((K, 128), jnp.float32),
            pltpu.VMEM((K, 128), jnp.float32),
            pltpu.VMEM((K, 128), jnp.float32),
            pltpu.VMEM((K, 128), jnp.float32),
            pltpu.VMEM_SHARED((NPAD, 128), jnp.float32),
            pltpu.SemaphoreType.DMA,
            pltpu.SemaphoreType.DMA,
            pltpu.SemaphoreType.DMA,
            pltpu.SemaphoreType.DMA,
            pltpu.SemaphoreType.DMA,
            pltpu.SemaphoreType.DMA,
            pltpu.SemaphoreType.DMA,
            pltpu.SemaphoreType.DMA,
        ],
    )
    def prop_kernel(hp_hbm, src_hbm, dst_hbm, zero_hbm, out_hbm,
                    src_v, dst_v, rows0, rows1, rows2, rows3, acc_sh,
                    semg0, semg1, semg2, semg3, semd0, semd1, semd2, semd3):
        cid = lax.axis_index("c")
        sid = lax.axis_index("s")
        rows = (rows0, rows1, rows2, rows3)
        semg = (semg0, semg1, semg2, semg3)
        semd = (semd0, semd1, semd2, semd3)
        hchunk = HROWS * 2

        @pl.when(cid == 0)
        def _():
            pltpu.sync_copy(zero_hbm,
                            acc_sh.at[pl.ds(sid * STRIPE, STRIPE)])
            plsc.subcore_barrier()

            for p in range(2):
                def src_view(c):
                    return src_v.at[c // 2, pl.ds((c % 2) * K, K)]

                def dst_hview(c, p=p):
                    return dst_hbm.at[sid, p * HROWS + c // 2,
                                      pl.ds((c % 2) * K, K)]

                def issue(c, b, p=p):
                    pltpu.async_copy(hp_hbm.at[src_view(c)], rows[b],
                                     semg[b])
                    pltpu.async_copy(dst_hview(c), dst_v.at[b], semd[b])

                pltpu.async_copy(
                    src_hbm.at[sid, pl.ds(p * HROWS, HROWS)], src_v,
                    semg0).wait()

                for b in range(DEPTH):
                    issue(b, b)

                @pl.loop(0, hchunk, step=DEPTH)
                def _(c, p=p):
                    for b in range(DEPTH):
                        pltpu.make_async_copy(
                            hp_hbm.at[src_view(c + b)], rows[b],
                            semg[b]).wait()
                        pltpu.make_async_copy(
                            dst_hview(c + b), dst_v.at[b], semd[b]).wait()
                        pltpu.sync_copy(rows[b], acc_sh.at[dst_v.at[b]],
                                        add=True)

                        @pl.when(c + b + DEPTH < hchunk)
                        def _(b=b):
                            issue(c + b + DEPTH, b)

            plsc.subcore_barrier()
            pltpu.sync_copy(acc_sh.at[pl.ds(sid * STRIPE, STRIPE)],
                            out_hbm.at[pl.ds(sid * STRIPE, STRIPE)])

    return prop_kernel(hp, srcp, dstp, zeros)


_RB = 1000


def _dis_block(degp_blk):
    deg = degp_blk[0, :, 0] + degp_blk[1, :, 0] + 1.0
    return lax.rsqrt(deg)[:, None]


def _mm1_body(x_ref, w_ref, degp_ref, o_ref):
    dis = _dis_block(degp_ref[...])
    h = jnp.dot(x_ref[...], w_ref[...], precision=lax.Precision.HIGHEST)
    o_ref[...] = dis * h


def _tc_mm1(x, W1, degp):
    return pl.pallas_call(
        _mm1_body,
        grid=(N // _RB,),
        in_specs=[
            pl.BlockSpec((_RB, 128), lambda i: (i, 0)),
            pl.BlockSpec((128, 128), lambda i: (0, 0)),
            pl.BlockSpec((NC, _RB, 16), lambda i: (0, i, 0)),
        ],
        out_specs=pl.BlockSpec((_RB, 128), lambda i: (i, 0)),
        out_shape=jax.ShapeDtypeStruct((N, 128), jnp.float32),
    )(x, W1, degp)


def _ep_mid_body(relu, acc_ref, hp_ref, degp_ref, b_ref, w_ref, o_ref):
    dis = _dis_block(degp_ref[...])
    t = dis * (acc_ref[...] + hp_ref[...]) + b_ref[...]
    if relu:
        t = jnp.maximum(t, 0.0)
    o_ref[...] = dis * jnp.dot(t, w_ref[...], precision=lax.Precision.HIGHEST)


def _tc_ep_mid(acc, hp, degp, b, W, relu, d_out):
    return pl.pallas_call(
        functools.partial(_ep_mid_body, relu),
        grid=(N // _RB,),
        in_specs=[
            pl.BlockSpec((_RB, 128), lambda i: (i, 0)),
            pl.BlockSpec((_RB, 128), lambda i: (i, 0)),
            pl.BlockSpec((NC, _RB, 16), lambda i: (0, i, 0)),
            pl.BlockSpec((1, 128), lambda i: (0, 0)),
            pl.BlockSpec((128, d_out), lambda i: (0, 0)),
        ],
        out_specs=pl.BlockSpec((_RB, d_out), lambda i: (i, 0)),
        out_shape=jax.ShapeDtypeStruct((N, d_out), jnp.float32),
    )(acc, hp, degp, b.reshape(1, 128), W)


def _ep3_body(acc_ref, hp_ref, degp_ref, b_ref, o_ref):
    dis = _dis_block(degp_ref[...])
    z = (dis * (acc_ref[...] + hp_ref[...]))[:, :64] + b_ref[...]
    m = jnp.max(z, axis=1, keepdims=True)
    s = z - m
    o_ref[...] = s - jnp.log(jnp.sum(jnp.exp(s), axis=1, keepdims=True))


def _tc_ep3(acc, hp, degp, b):
    return pl.pallas_call(
        _ep3_body,
        grid=(N // _RB,),
        in_specs=[
            pl.BlockSpec((_RB, 128), lambda i: (i, 0)),
            pl.BlockSpec((_RB, 128), lambda i: (i, 0)),
            pl.BlockSpec((NC, _RB, 16), lambda i: (0, i, 0)),
            pl.BlockSpec((1, 64), lambda i: (0, 0)),
        ],
        out_specs=pl.BlockSpec((_RB, 64), lambda i: (i, 0)),
        out_shape=jax.ShapeDtypeStruct((N, 64), jnp.float32),
    )(acc, hp, degp, b.reshape(1, 64))


def kernel(x, edge_index, W1, b1, W2, b2, W3, b3):
    srcp = _pack_edges(edge_index[0], 0)
    dstp = _pack_edges(edge_index[1], N)

    zeros16 = jnp.zeros((STRIPE, 16), jnp.float32)
    zeros128 = jnp.zeros((STRIPE, 128), jnp.float32)
    W3p = jnp.concatenate([W3, jnp.zeros((128, 64), jnp.float32)], axis=1)

    degp = _sc_degree(dstp, zeros16)

    h1 = _tc_mm1(x, W1, degp)
    acc1 = _sc_propagate(h1, srcp, dstp, zeros128)
    h2 = _tc_ep_mid(acc1, h1, degp, b1, W2, True, 128)
    acc2 = _sc_propagate(h2, srcp, dstp, zeros128)
    h3 = _tc_ep_mid(acc2, h2, degp, b2, W3p, False, 128)
    acc3 = _sc_propagate(h3, srcp, dstp, zeros128)
    return _tc_ep3(acc3, h3, degp, b3)


# speedup vs baseline: 1.9013x; 1.9013x over previous
import functools

import jax
import jax.numpy as jnp
from jax import lax
from jax.experimental import pallas as pl
from jax.experimental.pallas import tpu as pltpu
from jax.experimental.pallas import tpu_sc as plsc

N = 10000
E = 320000
NPAD = 10112
NC = 2
NS = 16
KB = 128
K = 64
NCH_ROWS = (128, 30)
NCH = (NCH_ROWS[0] * 2, NCH_ROWS[1] * 2)
NCMAX = max(NCH_ROWS)
STRIPE = NPAD // NS

_mesh = plsc.VectorSubcoreMesh(core_axis_name="c", subcore_axis_name="s")


def _split_edges(arr, pad_value):
    parts = []
    off = 0
    for c in range(NC):
        cap = NS * NCH_ROWS[c] * KB
        take = min(E - off, cap)
        blk = lax.dynamic_slice_in_dim(arr, off, take) if take > 0 else arr[:0]
        blk = jnp.concatenate(
            [blk, jnp.full((cap - take,), pad_value, jnp.int32)])
        blk = blk.reshape(NS, NCH_ROWS[c], KB)
        if NCH_ROWS[c] < NCMAX:
            blk = jnp.concatenate(
                [blk,
                 jnp.full((NS, NCMAX - NCH_ROWS[c], KB), pad_value,
                          jnp.int32)],
                axis=1)
        parts.append(blk)
        off += take
    return jnp.concatenate(parts, axis=0)


def _sc_degree(dstp, zeros16):
    @functools.partial(
        pl.kernel,
        out_type=jax.ShapeDtypeStruct((NC, NPAD, 16), jnp.float32),
        mesh=_mesh,
        scratch_types=[
            pltpu.VMEM((NCMAX, KB), jnp.int32),
            pltpu.VMEM((KB, 16), jnp.float32),
            pltpu.VMEM_SHARED((NPAD, 16), jnp.float32),
            pltpu.SemaphoreType.DMA,
        ],
    )
    def deg_kernel(dst_hbm, zero_hbm, out_hbm, dst_v, ones_v, acc_sh, sem):
        cid = lax.axis_index("c")
        sid = lax.axis_index("s")
        wid = cid * NS + sid

        pltpu.async_copy(dst_hbm.at[wid], dst_v, sem).wait()
        @pl.loop(0, KB)
        def _(r):
            ones_v[r, :] = jnp.full((16,), 1.0, jnp.float32)

        pltpu.sync_copy(zero_hbm.at[pl.ds(sid * STRIPE, STRIPE)],
                        acc_sh.at[pl.ds(sid * STRIPE, STRIPE)])
        plsc.subcore_barrier()

        for c_i in range(NC):
            @pl.when(cid == c_i)
            def _(c_i=c_i):
                @pl.loop(0, NCH_ROWS[c_i])
                def _(c):
                    pltpu.sync_copy(ones_v, acc_sh.at[dst_v.at[c]], add=True)

        plsc.subcore_barrier()
        pltpu.sync_copy(acc_sh.at[pl.ds(sid * STRIPE, STRIPE)],
                        out_hbm.at[cid, pl.ds(sid * STRIPE, STRIPE)])

    return deg_kernel(dstp, zeros16)


def _sc_propagate(hp, srcp, dstp, zeros):
    @functools.partial(
        pl.kernel,
        out_type=jax.ShapeDtypeStruct((NC, NPAD, 128), jnp.float32),
        mesh=_mesh,
        scratch_types=[
            pltpu.VMEM((NCMAX, KB), jnp.int32),
            pltpu.VMEM((4, K), jnp.int32),
            pltpu.VMEM((K, 128), jnp.float32),
            pltpu.VMEM((K, 128), jnp.float32),
            pltpu.VMEM((K, 128), jnp.float32),
            pltpu.VMEM((K, 128), jnp.float32),
            pltpu.VMEM_SHARED((NPAD, 128), jnp.float32),
            pltpu.SemaphoreType.DMA,
            pltpu.SemaphoreType.DMA,
            pltpu.SemaphoreType.DMA,
            pltpu.SemaphoreType.DMA,
            pltpu.SemaphoreType.DMA,
            pltpu.SemaphoreType.DMA,
            pltpu.SemaphoreType.DMA,
            pltpu.SemaphoreType.DMA,
        ],
    )
    def prop_kernel(hp_hbm, src_hbm, dst_hbm, zero_hbm, out_hbm,
                    src_v, dst_v, rows0, rows1, rows2, rows3, acc_sh,
                    semg0, semg1, semg2, semg3, semd0, semd1, semd2, semd3):
        cid = lax.axis_index("c")
        sid = lax.axis_index("s")
        wid = cid * NS + sid
        rows = (rows0, rows1, rows2, rows3)
        semg = (semg0, semg1, semg2, semg3)
        semd = (semd0, semd1, semd2, semd3)

        cp_s = pltpu.async_copy(src_hbm.at[wid], src_v, semg0)
        pltpu.sync_copy(zero_hbm.at[pl.ds(sid * STRIPE, STRIPE)],
                        acc_sh.at[pl.ds(sid * STRIPE, STRIPE)])
        cp_s.wait()
        plsc.subcore_barrier()

        def src_view(c):
            return src_v.at[c // 2, pl.ds((c % 2) * K, K)]

        def dst_hview(c):
            return dst_hbm.at[wid, c // 2, pl.ds((c % 2) * K, K)]

        def issue(c, b):
            pltpu.async_copy(hp_hbm.at[src_view(c)], rows[b], semg[b])
            pltpu.async_copy(dst_hview(c), dst_v.at[b], semd[b])

        for c_i in range(NC):
            nch = NCH[c_i]

            @pl.when(cid == c_i)
            def _(nch=nch):
                for b in range(4):
                    issue(b, b)

                @pl.loop(0, nch, step=4)
                def _(c):
                    for b in range(4):
                        pltpu.make_async_copy(
                            hp_hbm.at[src_view(c + b)], rows[b],
                            semg[b]).wait()
                        pltpu.make_async_copy(
                            dst_hview(c + b), dst_v.at[b],
                            semd[b]).wait()
                        pltpu.sync_copy(rows[b], acc_sh.at[dst_v.at[b]],
                                        add=True)

                        @pl.when(c + b + 4 < nch)
                        def _(b=b):
                            issue(c + b + 4, b)

        plsc.subcore_barrier()
        pltpu.sync_copy(acc_sh.at[pl.ds(sid * STRIPE, STRIPE)],
                        out_hbm.at[cid, pl.ds(sid * STRIPE, STRIPE)])

    return prop_kernel(hp, srcp, dstp, zeros)


_RB = 1000


def _dis_block(degp_blk):
    deg = degp_blk[0, :, 0] + degp_blk[1, :, 0] + 1.0
    return lax.rsqrt(deg)[:, None]


def _mm1_body(x_ref, w_ref, degp_ref, o_ref):
    dis = _dis_block(degp_ref[...])
    h = jnp.dot(x_ref[...], w_ref[...], precision=lax.Precision.HIGHEST)
    o_ref[...] = dis * h


def _tc_mm1(x, W1, degp):
    return pl.pallas_call(
        _mm1_body,
        grid=(N // _RB,),
        in_specs=[
            pl.BlockSpec((_RB, 128), lambda i: (i, 0)),
            pl.BlockSpec((128, 128), lambda i: (0, 0)),
            pl.BlockSpec((NC, _RB, 16), lambda i: (0, i, 0)),
        ],
        out_specs=pl.BlockSpec((_RB, 128), lambda i: (i, 0)),
        out_shape=jax.ShapeDtypeStruct((N, 128), jnp.float32),
    )(x, W1, degp)


def _ep_mid_body(relu, acc_ref, hp_ref, degp_ref, b_ref, w_ref, o_ref):
    dis = _dis_block(degp_ref[...])
    t = dis * (acc_ref[0] + acc_ref[1] + hp_ref[...]) + b_ref[...]
    if relu:
        t = jnp.maximum(t, 0.0)
    o_ref[...] = dis * jnp.dot(t, w_ref[...], precision=lax.Precision.HIGHEST)


def _tc_ep_mid(acc, hp, degp, b, W, relu, d_out):
    return pl.pallas_call(
        functools.partial(_ep_mid_body, relu),
        grid=(N // _RB,),
        in_specs=[
            pl.BlockSpec((NC, _RB, 128), lambda i: (0, i, 0)),
            pl.BlockSpec((_RB, 128), lambda i: (i, 0)),
            pl.BlockSpec((NC, _RB, 16), lambda i: (0, i, 0)),
            pl.BlockSpec((1, 128), lambda i: (0, 0)),
            pl.BlockSpec((128, d_out), lambda i: (0, 0)),
        ],
        out_specs=pl.BlockSpec((_RB, d_out), lambda i: (i, 0)),
        out_shape=jax.ShapeDtypeStruct((N, d_out), jnp.float32),
    )(acc, hp, degp, b.reshape(1, 128), W)


def _ep3_body(acc_ref, hp_ref, degp_ref, b_ref, o_ref):
    dis = _dis_block(degp_ref[...])
    z = (dis * (acc_ref[0] + acc_ref[1] + hp_ref[...]))[:, :64] + b_ref[...]
    m = jnp.max(z, axis=1, keepdims=True)
    s = z - m
    o_ref[...] = s - jnp.log(jnp.sum(jnp.exp(s), axis=1, keepdims=True))


def _tc_ep3(acc, hp, degp, b):
    return pl.pallas_call(
        _ep3_body,
        grid=(N // _RB,),
        in_specs=[
            pl.BlockSpec((NC, _RB, 128), lambda i: (0, i, 0)),
            pl.BlockSpec((_RB, 128), lambda i: (i, 0)),
            pl.BlockSpec((NC, _RB, 16), lambda i: (0, i, 0)),
            pl.BlockSpec((1, 64), lambda i: (0, 0)),
        ],
        out_specs=pl.BlockSpec((_RB, 64), lambda i: (i, 0)),
        out_shape=jax.ShapeDtypeStruct((N, 64), jnp.float32),
    )(acc, hp, degp, b.reshape(1, 64))


def kernel(x, edge_index, W1, b1, W2, b2, W3, b3):
    srcp = _split_edges(edge_index[0], 0)
    dstp = _split_edges(edge_index[1], N)

    zeros16 = jnp.zeros((NPAD, 16), jnp.float32)
    zeros128 = jnp.zeros((NPAD, 128), jnp.float32)
    W3p = jnp.concatenate([W3, jnp.zeros((128, 64), jnp.float32)], axis=1)

    degp = _sc_degree(dstp, zeros16)

    h1 = _tc_mm1(x, W1, degp)
    acc1 = _sc_propagate(h1, srcp, dstp, zeros128)
    h2 = _tc_ep_mid(acc1, h1, degp, b1, W2, True, 128)
    acc2 = _sc_propagate(h2, srcp, dstp, zeros128)
    h3 = _tc_ep_mid(acc2, h2, degp, b2, W3p, False, 128)
    acc3 = _sc_propagate(h3, srcp, dstp, zeros128)
    return _tc_ep3(acc3, h3, degp, b3)
